# Initial kernel scaffold; baseline (speedup 1.0000x reference)
import functools

import jax
import jax.numpy as jnp
from jax import lax
from jax.experimental import pallas as pl
from jax.experimental.pallas import tpu as pltpu
from jax.experimental.pallas import tpu_sc as plsc

_LANES = 128


@functools.cache
def _make_gather(n_rows: int, d: int, nw: int):
    rows_per_w = n_rows // nw
    mesh = plsc.VectorSubcoreMesh(core_axis_name="c", subcore_axis_name="s")

    @functools.partial(
        pl.kernel,
        mesh=mesh,
        out_type=jax.ShapeDtypeStruct((n_rows, _LANES, d), jnp.float32),
        scratch_types=[
            pltpu.VMEM((rows_per_w, _LANES), jnp.int32),
            pltpu.VMEM((_LANES, d), jnp.float32),
            pltpu.SemaphoreType.DMA,
        ],
    )
    def gather(table_hbm, idx_hbm, out_hbm, idx_v, rows_v, gsem):
        wid = lax.axis_index("s") * 2 + lax.axis_index("c")
        base = wid * rows_per_w
        pltpu.sync_copy(idx_hbm.at[pl.ds(base, rows_per_w)], idx_v)

        def step(j, carry):
            pltpu.async_copy(table_hbm.at[idx_v.at[j]], rows_v, gsem).wait()
            pltpu.sync_copy(rows_v, out_hbm.at[base + j])
            return carry

        lax.fori_loop(0, rows_per_w, step, 0)

    return gather


def kernel(vec, table):
    b, s = vec.shape
    v, d = table.shape
    total = b * s
    nw = 32
    assert total % (_LANES * nw) == 0
    n_rows = total // _LANES
    idx = vec.reshape(n_rows, _LANES).astype(jnp.int32)
    out = _make_gather(n_rows, d, nw)(table, idx)
    return out.reshape(b, s, d)


# speedup vs baseline: 2.9655x; 2.9655x over previous
import functools

import jax
import jax.numpy as jnp
from jax import lax
from jax.experimental import pallas as pl
from jax.experimental.pallas import tpu as pltpu
from jax.experimental.pallas import tpu_sc as plsc

_LANES = 128


@functools.cache
def _make_gather(n_rows: int, d: int, nw: int):
    rows_per_w = n_rows // nw
    mesh = plsc.VectorSubcoreMesh(core_axis_name="c", subcore_axis_name="s")

    @functools.partial(
        pl.kernel,
        mesh=mesh,
        out_type=jax.ShapeDtypeStruct((n_rows, _LANES, d), jnp.float32),
        scratch_types=[
            pltpu.VMEM((rows_per_w, _LANES), jnp.int32),
            pltpu.VMEM((_LANES, d), jnp.float32),
            pltpu.SemaphoreType.DMA,
        ],
    )
    def gather(table_hbm, idx_hbm, out_hbm, idx_v, rows_v, gsem):
        wid = lax.axis_index("s") * 2 + lax.axis_index("c")
        base = wid * rows_per_w
        pltpu.sync_copy(idx_hbm.at[wid], idx_v)

        def step(j, carry):
            pltpu.async_copy(table_hbm.at[idx_v.at[j]], rows_v, gsem).wait()
            pltpu.sync_copy(rows_v, out_hbm.at[base + j])
            return carry

        lax.fori_loop(0, rows_per_w, step, 0)

    return gather


def kernel(vec, table):
    b, s = vec.shape
    v, d = table.shape
    total = b * s
    nw = 32
    assert total % (_LANES * nw) == 0
    n_rows = total // _LANES
    idx = vec.reshape(nw, n_rows // nw, _LANES).astype(jnp.int32)
    out = _make_gather(n_rows, d, nw)(table, idx)
    return out.reshape(b, s, d)


# speedup vs baseline: 3.3290x; 1.1226x over previous
import functools

import jax
import jax.numpy as jnp
from jax import lax
from jax.experimental import pallas as pl
from jax.experimental.pallas import tpu as pltpu
from jax.experimental.pallas import tpu_sc as plsc

_LANES = 128


@functools.cache
def _make_gather(n_rows: int, d: int, nw: int):
    rows_per_w = n_rows // nw
    mesh = plsc.VectorSubcoreMesh(core_axis_name="c", subcore_axis_name="s")

    @functools.partial(
        pl.kernel,
        mesh=mesh,
        out_type=jax.ShapeDtypeStruct((n_rows, _LANES, d), jnp.float32),
        scratch_types=[
            pltpu.VMEM((rows_per_w, _LANES), jnp.int32),
            pltpu.VMEM((2, _LANES, d), jnp.float32),
            pltpu.SemaphoreType.DMA,
            pltpu.SemaphoreType.DMA,
            pltpu.SemaphoreType.DMA,
            pltpu.SemaphoreType.DMA,
        ],
    )
    def gather(table_hbm, idx_hbm, out_hbm, idx_v, rows_v, g0, g1, o0, o1):
        wid = lax.axis_index("s") * 2 + lax.axis_index("c")
        base = wid * rows_per_w
        gsem = (g0, g1)
        osem = (o0, o1)
        pltpu.sync_copy(idx_hbm.at[wid], idx_v)

        def start_gather(j, b):
            pltpu.async_copy(table_hbm.at[idx_v.at[j]], rows_v.at[b], gsem[b])

        def wait_gather(j, b):
            pltpu.make_async_copy(
                table_hbm.at[idx_v.at[j]], rows_v.at[b], gsem[b]).wait()

        def start_out(j, b):
            pltpu.async_copy(rows_v.at[b], out_hbm.at[base + j], osem[b])

        def wait_out(j, b):
            pltpu.make_async_copy(
                rows_v.at[b], out_hbm.at[base + j], osem[b]).wait()

        n_pairs = rows_per_w // 2
        start_gather(0, 0)

        def step(g, carry):
            for b in range(2):
                j = g * 2 + b
                nb = 1 - b
                if b == 0:
                    @pl.when(g > 0)
                    def _():
                        wait_out(j - 1, nb)
                        start_gather(j + 1, nb)

                    @pl.when(g == 0)
                    def _():
                        start_gather(j + 1, nb)
                else:
                    wait_out(j - 1, nb)

                    @pl.when(g < n_pairs - 1)
                    def _():
                        start_gather(j + 1, nb)
                wait_gather(j, b)
                start_out(j, b)
            return carry

        lax.fori_loop(0, n_pairs, step, 0)
        wait_out(rows_per_w - 1, 1)

    return gather


def kernel(vec, table):
    b, s = vec.shape
    v, d = table.shape
    total = b * s
    nw = 32
    assert total % (_LANES * nw) == 0
    n_rows = total // _LANES
    idx = vec.reshape(nw, n_rows // nw, _LANES).astype(jnp.int32)
    out = _make_gather(n_rows, d, nw)(table, idx)
    return out.reshape(b, s, d)


# speedup vs baseline: 3.3415x; 1.0038x over previous
import functools

import jax
import jax.numpy as jnp
from jax import lax
from jax.experimental import pallas as pl
from jax.experimental.pallas import tpu as pltpu
from jax.experimental.pallas import tpu_sc as plsc

_LANES = 128
_NBUF = 5
_AHEAD = 3


@functools.cache
def _make_gather(n_rows: int, d: int, nw: int):
    rows_per_w = n_rows // nw
    mesh = plsc.VectorSubcoreMesh(core_axis_name="c", subcore_axis_name="s")

    @functools.partial(
        pl.kernel,
        mesh=mesh,
        out_type=jax.ShapeDtypeStruct((n_rows, _LANES, d), jnp.float32),
        scratch_types=[
            pltpu.VMEM((rows_per_w, _LANES), jnp.int32),
            pltpu.VMEM((_NBUF, _LANES, d), jnp.float32),
        ] + [pltpu.SemaphoreType.DMA] * (2 * _NBUF),
    )
    def gather(table_hbm, idx_hbm, out_hbm, idx_v, rows_v, *sems):
        wid = lax.axis_index("s") * 2 + lax.axis_index("c")
        base = wid * rows_per_w
        gsem = sems[:_NBUF]
        osem = sems[_NBUF:]
        pltpu.sync_copy(idx_hbm.at[wid], idx_v)

        def start_gather(j, b):
            pltpu.async_copy(table_hbm.at[idx_v.at[j]], rows_v.at[b], gsem[b])

        def wait_gather(j, b):
            pltpu.make_async_copy(
                table_hbm.at[idx_v.at[j]], rows_v.at[b], gsem[b]).wait()

        def start_out(j, b):
            pltpu.async_copy(rows_v.at[b], out_hbm.at[base + j], osem[b])

        def wait_out(j, b):
            pltpu.make_async_copy(
                rows_v.at[b], out_hbm.at[base + j], osem[b]).wait()

        for b in range(_AHEAD):
            start_gather(b, b)

        n_outer = rows_per_w // _NBUF

        def step(g, carry):
            for b in range(_NBUF):
                j = g * _NBUF + b
                k_b = (b + _AHEAD) % _NBUF
                @pl.when(j + _AHEAD < rows_per_w)
                def _():
                    @pl.when(j + _AHEAD - _NBUF >= 0)
                    def _():
                        wait_out(j + _AHEAD - _NBUF, k_b)

                    start_gather(j + _AHEAD, k_b)

                wait_gather(j, b)
                start_out(j, b)
            return carry

        lax.fori_loop(0, n_outer, step, 0)
        for i in range(_NBUF):
            j = rows_per_w - _NBUF + i
            wait_out(j, j % _NBUF)

    return gather


def kernel(vec, table):
    b, s = vec.shape
    v, d = table.shape
    total = b * s
    nw = 32
    assert total % (_LANES * nw) == 0
    n_rows = total // _LANES
    idx = vec.reshape(nw, n_rows // nw, _LANES).astype(jnp.int32)
    out = _make_gather(n_rows, d, nw)(table, idx)
    return out.reshape(b, s, d)


# speedup vs baseline: 5.9585x; 1.7832x over previous
import functools

import jax
import jax.numpy as jnp
from jax import lax
from jax.experimental import pallas as pl
from jax.experimental.pallas import tpu as pltpu
from jax.experimental.pallas import tpu_sc as plsc

_NBUF = 8
_AHEAD = 4


@functools.cache
def _make_gather(b: int, s: int, d: int, nw: int):
    rows_per_w = b // nw
    mesh = plsc.VectorSubcoreMesh(core_axis_name="c", subcore_axis_name="s")

    @functools.partial(
        pl.kernel,
        mesh=mesh,
        out_type=jax.ShapeDtypeStruct((b, s, d), jnp.float32),
        scratch_types=[
            pltpu.VMEM((rows_per_w, s), jnp.int32),
            pltpu.VMEM((_NBUF, s, d), jnp.float32),
        ] + [pltpu.SemaphoreType.DMA] * (2 * _NBUF),
    )
    def gather(table_hbm, idx_hbm, out_hbm, idx_v, rows_v, *sems):
        wid = lax.axis_index("s") * 2 + lax.axis_index("c")
        base = wid * rows_per_w
        gsem = sems[:_NBUF]
        osem = sems[_NBUF:]
        pltpu.sync_copy(idx_hbm.at[pl.ds(base, rows_per_w)], idx_v)

        def start_gather(j, bf):
            pltpu.async_copy(table_hbm.at[idx_v.at[j]], rows_v.at[bf], gsem[bf])

        def wait_gather(j, bf):
            pltpu.make_async_copy(
                table_hbm.at[idx_v.at[j]], rows_v.at[bf], gsem[bf]).wait()

        def start_out(j, bf):
            pltpu.async_copy(rows_v.at[bf], out_hbm.at[base + j], osem[bf])

        def wait_out(j, bf):
            pltpu.make_async_copy(
                rows_v.at[bf], out_hbm.at[base + j], osem[bf]).wait()

        for bf in range(_AHEAD):
            start_gather(bf, bf)

        def step(g, carry):
            for bf in range(_NBUF):
                j = g * _NBUF + bf
                k_b = (bf + _AHEAD) % _NBUF
                @pl.when(j + _AHEAD < rows_per_w)
                def _():
                    @pl.when(j + _AHEAD - _NBUF >= 0)
                    def _():
                        wait_out(j + _AHEAD - _NBUF, k_b)

                    start_gather(j + _AHEAD, k_b)

                wait_gather(j, bf)
                start_out(j, bf)
            return carry

        lax.fori_loop(0, rows_per_w // _NBUF, step, 0)
        for i in range(_NBUF):
            j = rows_per_w - _NBUF + i
            wait_out(j, j % _NBUF)

    return gather


def kernel(vec, table):
    b, s = vec.shape
    v, d = table.shape
    nw = 32
    assert b % (nw * _NBUF) == 0
    idx = vec.astype(jnp.int32)
    return _make_gather(b, s, d, nw)(table, idx)


# speedup vs baseline: 5.9774x; 1.0032x over previous
import functools

import jax
import jax.numpy as jnp
from jax import lax
from jax.experimental import pallas as pl
from jax.experimental.pallas import tpu as pltpu
from jax.experimental.pallas import tpu_sc as plsc

_NBUF = 8
_AHEAD = 4


@functools.cache
def _make_gather(b: int, s: int, d: int, nw: int):
    rows_per_w = b // nw
    mesh = plsc.VectorSubcoreMesh(core_axis_name="c", subcore_axis_name="s")

    @functools.partial(
        pl.kernel,
        mesh=mesh,
        out_type=jax.ShapeDtypeStruct((b, s, d), jnp.float32),
        scratch_types=[
            pltpu.VMEM((rows_per_w, s), jnp.int32),
            pltpu.VMEM((_NBUF, s, d), jnp.float32),
        ] + [pltpu.SemaphoreType.DMA] * (2 * _NBUF),
        compiler_params=pltpu.CompilerParams(use_tc_tiling_on_sc=True),
    )
    def gather(table_hbm, idx_hbm, out_hbm, idx_v, rows_v, *sems):
        wid = lax.axis_index("s") * 2 + lax.axis_index("c")
        base = wid * rows_per_w
        gsem = sems[:_NBUF]
        osem = sems[_NBUF:]
        pltpu.sync_copy(idx_hbm.at[pl.ds(base, rows_per_w)], idx_v)

        def start_gather(j, bf):
            pltpu.async_copy(table_hbm.at[idx_v.at[j]], rows_v.at[bf], gsem[bf])

        def wait_gather(j, bf):
            pltpu.make_async_copy(
                table_hbm.at[idx_v.at[j]], rows_v.at[bf], gsem[bf]).wait()

        def start_out(j, bf):
            pltpu.async_copy(rows_v.at[bf], out_hbm.at[base + j], osem[bf])

        def wait_out(j, bf):
            pltpu.make_async_copy(
                rows_v.at[bf], out_hbm.at[base + j], osem[bf]).wait()

        for bf in range(_AHEAD):
            start_gather(bf, bf)

        def step(g, carry):
            for bf in range(_NBUF):
                j = g * _NBUF + bf
                k_b = (bf + _AHEAD) % _NBUF
                @pl.when(j + _AHEAD < rows_per_w)
                def _():
                    @pl.when(j + _AHEAD - _NBUF >= 0)
                    def _():
                        wait_out(j + _AHEAD - _NBUF, k_b)

                    start_gather(j + _AHEAD, k_b)

                wait_gather(j, bf)
                start_out(j, bf)
            return carry

        lax.fori_loop(0, rows_per_w // _NBUF, step, 0)
        for i in range(_NBUF):
            j = rows_per_w - _NBUF + i
            wait_out(j, j % _NBUF)

    return gather


def kernel(vec, table):
    b, s = vec.shape
    v, d = table.shape
    nw = 32
    assert b % (nw * _NBUF) == 0
    idx = vec.astype(jnp.int32)
    return _make_gather(b, s, d, nw)(table, idx)


# speedup vs baseline: 10.4916x; 1.7552x over previous
import functools

import jax
import jax.numpy as jnp
from jax import lax
from jax.experimental import pallas as pl
from jax.experimental.pallas import tpu as pltpu
from jax.experimental.pallas import tpu_sc as plsc

_LANES = 128
_NBUF = 5
_AHEAD = 3


@functools.cache
def _make_gather(n_rows: int, d: int, nw: int):
    rows_per_w = n_rows // nw
    mesh = plsc.VectorSubcoreMesh(core_axis_name="c", subcore_axis_name="s")

    @functools.partial(
        pl.kernel,
        mesh=mesh,
        out_type=jax.ShapeDtypeStruct((n_rows, _LANES, d), jnp.float32),
        scratch_types=[
            pltpu.VMEM((rows_per_w, _LANES), jnp.int32),
            pltpu.VMEM((_NBUF, _LANES, d), jnp.float32),
        ] + [pltpu.SemaphoreType.DMA] * (2 * _NBUF),
    )
    def gather(table_hbm, idx_hbm, out_hbm, idx_v, rows_v, *sems):
        wid = lax.axis_index("s") * 2 + lax.axis_index("c")
        base = wid * rows_per_w
        gsem = sems[:_NBUF]
        osem = sems[_NBUF:]
        pltpu.sync_copy(idx_hbm.at[wid], idx_v)

        def start_gather(j, bf):
            pltpu.async_copy(table_hbm.at[idx_v.at[j]], rows_v.at[bf], gsem[bf])

        def wait_gather(j, bf):
            pltpu.make_async_copy(
                table_hbm.at[idx_v.at[j]], rows_v.at[bf], gsem[bf]).wait()

        def start_out(j, bf):
            pltpu.async_copy(rows_v.at[bf], out_hbm.at[base + j], osem[bf])

        def wait_out(j, bf):
            pltpu.make_async_copy(
                rows_v.at[bf], out_hbm.at[base + j], osem[bf]).wait()

        for bf in range(_AHEAD):
            start_gather(bf, bf)

        def step(g, carry):
            for bf in range(_NBUF):
                j = g * _NBUF + bf
                k_b = (bf + _AHEAD) % _NBUF
                @pl.when(j + _AHEAD < rows_per_w)
                def _():
                    @pl.when(j + _AHEAD - _NBUF >= 0)
                    def _():
                        wait_out(j + _AHEAD - _NBUF, k_b)

                    start_gather(j + _AHEAD, k_b)

                wait_gather(j, bf)
                start_out(j, bf)
            return carry

        lax.fori_loop(0, rows_per_w // _NBUF, step, 0)
        for i in range(_NBUF):
            j = rows_per_w - _NBUF + i
            wait_out(j, j % _NBUF)

    return gather


def kernel(vec, table):
    b, s = vec.shape
    v, d = table.shape
    total = b * s
    nw = 32
    assert total % (_LANES * nw) == 0
    n_rows = total // _LANES
    idx = vec.T.reshape(nw, n_rows // nw, _LANES).astype(jnp.int32)
    out = _make_gather(n_rows, d, nw)(table, idx)
    return out.reshape(s, b, d).transpose(1, 0, 2)
